# baseline (speedup 1.0000x reference)
import functools

import jax
import jax.numpy as jnp
from jax import lax
from jax.experimental import pallas as pl
from jax.experimental.pallas import tpu as pltpu
from jax.experimental.pallas import tpu_sc as plsc

L = 16
NW = 16
CH = 16000
VCH = CH // L
SBINS = 1024

_GDN = lax.GatherDimensionNumbers(
    offset_dims=(), collapsed_slice_dims=(0,), start_index_map=(0,))


def _vtake(x, i):
    return lax.gather(x, i[:, None], _GDN, (1,),
                      mode=lax.GatherScatterMode.PROMISE_IN_BOUNDS)


def _ind01(x):
    return jnp.minimum(jnp.maximum(x, 0), 1)


def _prefix_sum(x, io):
    for s in (1, 2, 4, 8):
        x = x + _vtake(x, jnp.maximum(io - s, 0)) * _ind01(io - (s - 1))
    return x


def _tree_splat_sum(x, io):
    for s in (1, 2, 4, 8):
        x = x + _vtake(x, jnp.bitwise_xor(io, s))
    return x


def _rsqrt(y):
    r = 1.28 - 0.29 * y
    for _ in range(3):
        r = r * (1.5 - 0.5 * y * r * r)
    return r


def _body(n_sel, t_max, n_pairs_c,
          q_hbm, pair_hbm, d_hbm, asi_hbm, out_hbm,
          ii_v, jj_v, d_v, qi_v, qj_v, val_v, aidx_v, seg_v,
          acc_v, cnt_v, base_v, allcnt_v, shared_cnt, acc_sh,
          sem1, sem2, sem3):
    w = lax.axis_index("s")
    io = lax.iota(jnp.int32, L)
    zf = jnp.zeros((L,), jnp.float32)
    nspl = jnp.full((L,), n_sel, jnp.int32)

    zw = SBINS // NW
    for k in range(zw // L):
        acc_v[pl.ds(k * L, L)] = zf
    pltpu.sync_copy(acc_v.at[pl.ds(0, zw)], acc_sh.at[pl.ds(w * zw, zw)])
    plsc.subcore_barrier()

    def super_body(t, r_glob):
        run01 = jnp.minimum(jnp.maximum(n_sel - r_glob, 0), 1)
        e0 = t * (NW * CH) + w * CH

        def chunk(_, c):
            pltpu.sync_copy(pair_hbm.at[pl.ds(e0, CH)], ii_v)
            pltpu.sync_copy(pair_hbm.at[pl.ds(n_pairs_c + e0, CH)], jj_v)
            cqi = pltpu.async_copy(q_hbm.at[ii_v], qi_v, sem1)
            cqj = pltpu.async_copy(q_hbm.at[jj_v], qj_v, sem2)
            cd = pltpu.async_copy(d_hbm.at[pl.ds(e0, CH)], d_v, sem3)

            cnt_v[...] = jnp.zeros((L,), jnp.int32)

            def p1(k, c1):
                a = ii_v[pl.ds(k * L, L)]
                b = jj_v[pl.ds(k * L, L)]
                cnt_v[...] = cnt_v[...] + _ind01(b - a)
                return c1
            lax.fori_loop(0, VCH, p1, 0)
            cnt_v[...] = _tree_splat_sum(cnt_v[...], io)

            pltpu.sync_copy(cnt_v, shared_cnt.at[pl.ds(w * L, L)])
            plsc.subcore_barrier()
            pltpu.sync_copy(shared_cnt, allcnt_v)
            plsc.subcore_barrier()

            exv = jnp.zeros((L,), jnp.int32)
            tov = jnp.zeros((L,), jnp.int32)
            for v in range(NW):
                row = allcnt_v[pl.ds(v * L, L)]
                exv = exv + row * jnp.minimum(jnp.maximum(w - v, 0), 1)
                tov = tov + row
            mybase = exv[0] + r_glob
            cnt_v[...] = tov

            cqi.wait()
            cqj.wait()
            cd.wait()

            g2 = jnp.minimum(jnp.maximum(n_sel - mybase, 0), 1)
            mb_spl = jnp.full((L,), mybase, jnp.int32)
            base_v[...] = jnp.zeros((L,), jnp.int32)

            def p2(k, c2):
                bv = base_v[...]
                a = ii_v[pl.ds(k * L, L)]
                b = jj_v[pl.ds(k * L, L)]
                dd = d_v[pl.ds(k * L, L)]
                qi = qi_v[pl.ds(k * L, L)]
                qj = qj_v[pl.ds(k * L, L)]
                mi = _ind01(b - a)
                inc = _prefix_sum(mi, io)
                grank = bv + inc - mi + mb_spl
                act = (mi * _ind01(nspl - grank)).astype(jnp.float32)
                dc = jnp.maximum(dd, 1e-35)
                x = dd * 4.0
                poly = 1.0 + x * x * x * (-10.0 + x * (15.0 - 6.0 * x))
                inside = jnp.maximum(jnp.sign(0.25 - dd), 0.0)
                phi = poly * inside
                chi = phi * _rsqrt(dd * dd + 1.0) + (1.0 - phi) / dc
                val_v[pl.ds(k * L, L)] = qi * qj * chi * act
                aidx_v[pl.ds(k * L, L)] = jnp.minimum(grank, nspl - 1)
                base_v[...] = bv + _vtake(inc, jnp.full((L,), L - 1,
                                                        jnp.int32))
                return c2
            lax.fori_loop(0, VCH * g2, p2, 0)

            def dma_out(_, c3):
                pltpu.async_copy(asi_hbm.at[aidx_v], seg_v, sem1).wait()
                pltpu.async_copy(val_v, acc_sh.at[seg_v], sem2,
                                 add=True).wait()
                return c3
            lax.fori_loop(0, g2, dma_out, 0)
            return c

        cnt_v[...] = jnp.zeros((L,), jnp.int32)
        lax.fori_loop(0, run01, chunk, 0)
        return r_glob + cnt_v[...][0]

    lax.fori_loop(0, t_max, super_body, jnp.int32(0))

    plsc.subcore_barrier()
    pltpu.sync_copy(acc_sh, acc_v)
    for k in range(SBINS // L):
        acc_v[pl.ds(k * L, L)] = acc_v[pl.ds(k * L, L)] * 138.96
    pltpu.sync_copy(acc_v, out_hbm)


def kernel(per_atom_charge, d_ij, per_system_energy,
           atomic_subsystem_indices, pair_indices):
    n_atoms = per_atom_charge.shape[0]
    n_pairs = pair_indices.shape[1]
    n_sys = per_system_energy.shape[0]
    assert n_pairs % (NW * CH) == 0
    t_max = n_pairs // (NW * CH)

    q = per_atom_charge.astype(jnp.float32)
    d = d_ij.reshape(n_pairs).astype(jnp.float32)
    pairs = pair_indices.astype(jnp.int32).reshape(2 * n_pairs)
    asi = atomic_subsystem_indices.astype(jnp.int32)

    mesh = plsc.VectorSubcoreMesh(
        core_axis_name="c", subcore_axis_name="s", num_cores=1)
    fn = pl.kernel(
        functools.partial(_body, n_atoms, t_max, n_pairs),
        out_type=jax.ShapeDtypeStruct((SBINS,), jnp.float32),
        mesh=mesh,
        scratch_types=[
            pltpu.VMEM((CH,), jnp.int32),
            pltpu.VMEM((CH,), jnp.int32),
            pltpu.VMEM((CH,), jnp.float32),
            pltpu.VMEM((CH,), jnp.float32),
            pltpu.VMEM((CH,), jnp.float32),
            pltpu.VMEM((CH,), jnp.float32),
            pltpu.VMEM((CH,), jnp.int32),
            pltpu.VMEM((CH,), jnp.int32),
            pltpu.VMEM((SBINS,), jnp.float32),
            pltpu.VMEM((L,), jnp.int32),
            pltpu.VMEM((L,), jnp.int32),
            pltpu.VMEM((NW * L,), jnp.int32),
            pltpu.VMEM_SHARED((NW * L,), jnp.int32),
            pltpu.VMEM_SHARED((SBINS,), jnp.float32),
            pltpu.SemaphoreType.DMA,
            pltpu.SemaphoreType.DMA,
            pltpu.SemaphoreType.DMA,
        ],
    )
    out = fn(q, pairs, d, asi)
    proj = jnp.eye(SBINS, n_sys, dtype=jnp.float32)
    return (out @ proj).astype(per_system_energy.dtype)


# speedup vs baseline: 1.0720x; 1.0720x over previous
import functools

import jax
import jax.numpy as jnp
from jax import lax
from jax.experimental import pallas as pl
from jax.experimental.pallas import tpu as pltpu
from jax.experimental.pallas import tpu_sc as plsc

L = 16
NW = 16
CH = 8000
VCH = CH // L
SBINS = 1024

_GDN = lax.GatherDimensionNumbers(
    offset_dims=(), collapsed_slice_dims=(0,), start_index_map=(0,))


def _vtake(x, i):
    return lax.gather(x, i[:, None], _GDN, (1,),
                      mode=lax.GatherScatterMode.PROMISE_IN_BOUNDS)


def _ind01(x):
    return jnp.minimum(jnp.maximum(x, 0), 1)


def _prefix_sum(x, io):
    for s in (1, 2, 4, 8):
        x = x + _vtake(x, jnp.maximum(io - s, 0)) * _ind01(io - (s - 1))
    return x


def _tree_splat_sum(x, io):
    for s in (1, 2, 4, 8):
        x = x + _vtake(x, jnp.bitwise_xor(io, s))
    return x


def _rsqrt(y):
    r = 1.28 - 0.29 * y
    for _ in range(3):
        r = r * (1.5 - 0.5 * y * r * r)
    return r


def _body(n_sel, t_max, n_pairs_c,
          q_hbm, pair_hbm, d_hbm, asi_hbm, out_hbm,
          ii_v, jj_v, d_v, qi_v, qj_v, val_v, aidx_v, seg_v,
          acc_v, cnt_v, base_v, allcnt_v, shared_cnt, acc_sh,
          sem1, sem2, sem3):
    w = lax.axis_index("s")
    io = lax.iota(jnp.int32, L)
    zf = jnp.zeros((L,), jnp.float32)
    nspl = jnp.full((L,), n_sel, jnp.int32)

    zw = SBINS // NW
    for k in range(zw // L):
        acc_v[pl.ds(k * L, L)] = zf
    pltpu.sync_copy(acc_v.at[pl.ds(0, zw)], acc_sh.at[pl.ds(w * zw, zw)])
    plsc.subcore_barrier()

    def super_body(t, r_glob):
        run01 = jnp.minimum(jnp.maximum(n_sel - r_glob, 0), 1)
        e0 = t * (NW * CH) + w * CH

        def chunk(_, c):
            pltpu.sync_copy(pair_hbm.at[pl.ds(e0, CH)], ii_v)
            pltpu.sync_copy(pair_hbm.at[pl.ds(n_pairs_c + e0, CH)], jj_v)
            cqi = pltpu.async_copy(q_hbm.at[ii_v], qi_v, sem1)
            cqj = pltpu.async_copy(q_hbm.at[jj_v], qj_v, sem2)
            cd = pltpu.async_copy(d_hbm.at[pl.ds(e0, CH)], d_v, sem3)

            cnt_v[...] = jnp.zeros((L,), jnp.int32)

            def p1(k, c1):
                a = ii_v[pl.ds(k * L, L)]
                b = jj_v[pl.ds(k * L, L)]
                cnt_v[...] = cnt_v[...] + _ind01(b - a)
                return c1
            lax.fori_loop(0, VCH, p1, 0)
            cnt_v[...] = _tree_splat_sum(cnt_v[...], io)

            pltpu.sync_copy(cnt_v, shared_cnt.at[pl.ds(w * L, L)])
            plsc.subcore_barrier()
            pltpu.sync_copy(shared_cnt, allcnt_v)
            plsc.subcore_barrier()

            exv = jnp.zeros((L,), jnp.int32)
            tov = jnp.zeros((L,), jnp.int32)
            for v in range(NW):
                row = allcnt_v[pl.ds(v * L, L)]
                exv = exv + row * jnp.minimum(jnp.maximum(w - v, 0), 1)
                tov = tov + row
            mybase = exv[0] + r_glob
            cnt_v[...] = tov

            cqi.wait()
            cqj.wait()
            cd.wait()

            g2 = jnp.minimum(jnp.maximum(n_sel - mybase, 0), 1)
            mb_spl = jnp.full((L,), mybase, jnp.int32)
            base_v[...] = jnp.zeros((L,), jnp.int32)

            def p2(k, c2):
                bv = base_v[...]
                a = ii_v[pl.ds(k * L, L)]
                b = jj_v[pl.ds(k * L, L)]
                dd = d_v[pl.ds(k * L, L)]
                qi = qi_v[pl.ds(k * L, L)]
                qj = qj_v[pl.ds(k * L, L)]
                mi = _ind01(b - a)
                inc = _prefix_sum(mi, io)
                grank = bv + inc - mi + mb_spl
                act = (mi * _ind01(nspl - grank)).astype(jnp.float32)
                dc = jnp.maximum(dd, 1e-35)
                x = dd * 4.0
                poly = 1.0 + x * x * x * (-10.0 + x * (15.0 - 6.0 * x))
                inside = jnp.maximum(jnp.sign(0.25 - dd), 0.0)
                phi = poly * inside
                chi = phi * _rsqrt(dd * dd + 1.0) + (1.0 - phi) / dc
                val_v[pl.ds(k * L, L)] = qi * qj * chi * act
                aidx_v[pl.ds(k * L, L)] = jnp.minimum(grank, nspl - 1)
                base_v[...] = bv + _vtake(inc, jnp.full((L,), L - 1,
                                                        jnp.int32))
                return c2
            lax.fori_loop(0, VCH * g2, p2, 0)

            def dma_out(_, c3):
                pltpu.async_copy(asi_hbm.at[aidx_v], seg_v, sem1).wait()
                pltpu.async_copy(val_v, acc_sh.at[seg_v], sem2,
                                 add=True).wait()
                return c3
            lax.fori_loop(0, g2, dma_out, 0)
            return c

        cnt_v[...] = jnp.zeros((L,), jnp.int32)
        lax.fori_loop(0, run01, chunk, 0)
        return r_glob + cnt_v[...][0]

    lax.fori_loop(0, t_max, super_body, jnp.int32(0))

    plsc.subcore_barrier()
    pltpu.sync_copy(acc_sh, acc_v)
    for k in range(SBINS // L):
        acc_v[pl.ds(k * L, L)] = acc_v[pl.ds(k * L, L)] * 138.96
    pltpu.sync_copy(acc_v, out_hbm)


def kernel(per_atom_charge, d_ij, per_system_energy,
           atomic_subsystem_indices, pair_indices):
    n_atoms = per_atom_charge.shape[0]
    n_pairs = pair_indices.shape[1]
    n_sys = per_system_energy.shape[0]
    assert n_pairs % (NW * CH) == 0
    t_max = n_pairs // (NW * CH)

    q = per_atom_charge.astype(jnp.float32)
    d = d_ij.reshape(n_pairs).astype(jnp.float32)
    pairs = pair_indices.astype(jnp.int32).reshape(2 * n_pairs)
    asi = atomic_subsystem_indices.astype(jnp.int32)

    mesh = plsc.VectorSubcoreMesh(
        core_axis_name="c", subcore_axis_name="s", num_cores=1)
    fn = pl.kernel(
        functools.partial(_body, n_atoms, t_max, n_pairs),
        out_type=jax.ShapeDtypeStruct((SBINS,), jnp.float32),
        mesh=mesh,
        scratch_types=[
            pltpu.VMEM((CH,), jnp.int32),
            pltpu.VMEM((CH,), jnp.int32),
            pltpu.VMEM((CH,), jnp.float32),
            pltpu.VMEM((CH,), jnp.float32),
            pltpu.VMEM((CH,), jnp.float32),
            pltpu.VMEM((CH,), jnp.float32),
            pltpu.VMEM((CH,), jnp.int32),
            pltpu.VMEM((CH,), jnp.int32),
            pltpu.VMEM((SBINS,), jnp.float32),
            pltpu.VMEM((L,), jnp.int32),
            pltpu.VMEM((L,), jnp.int32),
            pltpu.VMEM((NW * L,), jnp.int32),
            pltpu.VMEM_SHARED((NW * L,), jnp.int32),
            pltpu.VMEM_SHARED((SBINS,), jnp.float32),
            pltpu.SemaphoreType.DMA,
            pltpu.SemaphoreType.DMA,
            pltpu.SemaphoreType.DMA,
        ],
    )
    out = fn(q, pairs, d, asi)
    proj = jnp.eye(SBINS, n_sys, dtype=jnp.float32)
    return (out @ proj).astype(per_system_energy.dtype)
